# baseline (speedup 1.0000x reference)
import functools
import jax
import jax.numpy as jnp
from jax import lax
from jax.experimental import pallas as pl
from jax.experimental.pallas import tpu as pltpu
from jax.experimental.pallas import tpu_sc as plsc

N = 10000
E = 320000
D = 128
EW = 16

NC = 2
NS = 16
NWORK = NC * NS
EPW = E // NWORK
K = 80
G = 5
SUP = G * K
NSUP = EPW // SUP
RPT = N // NS

f32 = jnp.float32
i32 = jnp.int32

_MESH = plsc.VectorSubcoreMesh(core_axis_name="c", subcore_axis_name="s")
_SC_PARAMS = pltpu.CompilerParams(needs_layout_passes=False, use_tc_tiling_on_sc=False)


def _lrelu(x):
    return jnp.maximum(x, 0.1 * x)


def _tc1a_body(h_ref, wen_ref, wnsa_ref, bias_ref, hn_ref, asad_ref):
    hn = jnp.dot(h_ref[...], wen_ref[...], preferred_element_type=f32)
    hn_ref[...] = hn
    asad_ref[...] = jnp.dot(hn, wnsa_ref[...], preferred_element_type=f32) + bias_ref[...]


def _tc1a(h, Wen, Wnsa2, bias2):
    nb = 10
    bs = N // nb
    return pl.pallas_call(
        _tc1a_body,
        grid=(nb,),
        in_specs=[
            pl.BlockSpec((bs, D), lambda i: (i, 0)),
            pl.BlockSpec((D, D), lambda i: (0, 0)),
            pl.BlockSpec((D, 2), lambda i: (0, 0)),
            pl.BlockSpec((1, 2), lambda i: (0, 0)),
        ],
        out_specs=[
            pl.BlockSpec((bs, D), lambda i: (i, 0)),
            pl.BlockSpec((bs, 2), lambda i: (i, 0)),
        ],
        out_shape=[
            jax.ShapeDtypeStruct((N, D), f32),
            jax.ShapeDtypeStruct((N, 2), f32),
        ],
    )(h, Wen, Wnsa2, bias2)


def _tc1b_body(ew_ref, wee_ref, wa3_ref, w_ref, wa3o_ref):
    w = jnp.dot(ew_ref[...], wee_ref[...], preferred_element_type=f32)
    w_ref[...] = w
    wa3o_ref[...] = jnp.dot(w, wa3_ref[...], preferred_element_type=f32)


def _tc1b(edge_w, Wee, Wa3):
    nb = 80
    bs = E // nb
    return pl.pallas_call(
        _tc1b_body,
        grid=(nb,),
        in_specs=[
            pl.BlockSpec((bs, EW), lambda i: (i, 0)),
            pl.BlockSpec((EW, EW), lambda i: (0, 0)),
            pl.BlockSpec((EW, 1), lambda i: (0, 0)),
        ],
        out_specs=[
            pl.BlockSpec((bs, EW), lambda i: (i, 0)),
            pl.BlockSpec((bs, 1), lambda i: (i, 0)),
        ],
        out_shape=[
            jax.ShapeDtypeStruct((E, EW), f32),
            jax.ShapeDtypeStruct((E, 1), f32),
        ],
    )(edge_w, Wee, Wa3)


def _tc2_body(parts_ref, recip_ref, wat_ref, h1_ref, b12_ref):
    h1 = (parts_ref[0] + parts_ref[1]) * recip_ref[...]
    h1_ref[...] = h1
    b12_ref[...] = jnp.dot(h1, wat_ref[...], preferred_element_type=f32)


def _tc2(parts, recip, Wat12):
    nb = 10
    bs = N // nb
    return pl.pallas_call(
        _tc2_body,
        grid=(nb,),
        in_specs=[
            pl.BlockSpec((NC, bs, D), lambda i: (0, i, 0)),
            pl.BlockSpec((bs, 1), lambda i: (i, 0)),
            pl.BlockSpec((D, 2), lambda i: (0, 0)),
        ],
        out_specs=[
            pl.BlockSpec((bs, D), lambda i: (i, 0)),
            pl.BlockSpec((bs, 2), lambda i: (i, 0)),
        ],
        out_shape=[
            jax.ShapeDtypeStruct((N, D), f32),
            jax.ShapeDtypeStruct((N, 2), f32),
        ],
    )(parts, recip, Wat12)


def _tc3_body(ph_ref, pw_ref, recip_ref, hn_ref, wc1_ref, wc2_ref,
              wc3_ref, bc_ref, wa1_ref, wa2_ref, h2_ref, p1_ref, p2_ref):
    r = recip_ref[...]
    haggH = (ph_ref[0] + ph_ref[1]) * r
    haggW = (pw_ref[0] + pw_ref[1]) * r
    h2 = (jnp.dot(haggH, wc1_ref[...], preferred_element_type=f32)
          + jnp.dot(haggW, wc2_ref[...], preferred_element_type=f32)
          + jnp.dot(hn_ref[...], wc3_ref[...], preferred_element_type=f32)
          + bc_ref[...])
    h2_ref[...] = h2
    p1_ref[...] = jnp.dot(h2, wa1_ref[...], preferred_element_type=f32)
    p2_ref[...] = jnp.dot(h2, wa2_ref[...], preferred_element_type=f32)


def _tc3(ph, pw, recip, hn, Wc1, Wc2, Wc3, bc, Wa1, Wa2):
    nb = 10
    bs = N // nb
    return pl.pallas_call(
        _tc3_body,
        grid=(nb,),
        in_specs=[
            pl.BlockSpec((NC, bs, D), lambda i: (0, i, 0)),
            pl.BlockSpec((NC, bs, EW), lambda i: (0, i, 0)),
            pl.BlockSpec((bs, 1), lambda i: (i, 0)),
            pl.BlockSpec((bs, D), lambda i: (i, 0)),
            pl.BlockSpec((D, D), lambda i: (0, 0)),
            pl.BlockSpec((EW, D), lambda i: (0, 0)),
            pl.BlockSpec((D, D), lambda i: (0, 0)),
            pl.BlockSpec((1, D), lambda i: (0, 0)),
            pl.BlockSpec((D, EW), lambda i: (0, 0)),
            pl.BlockSpec((D, EW), lambda i: (0, 0)),
        ],
        out_specs=[
            pl.BlockSpec((bs, D), lambda i: (i, 0)),
            pl.BlockSpec((bs, EW), lambda i: (i, 0)),
            pl.BlockSpec((bs, EW), lambda i: (i, 0)),
        ],
        out_shape=[
            jax.ShapeDtypeStruct((N, D), f32),
            jax.ShapeDtypeStruct((N, EW), f32),
            jax.ShapeDtypeStruct((N, EW), f32),
        ],
    )(ph, pw, recip, hn, Wc1, Wc2, Wc3, bc, Wa1, Wa2)


def _tc4a_body(w_ref, stats_ref):
    i = pl.program_id(0)

    @pl.when(i == 0)
    def _():
        stats_ref[...] = jnp.zeros_like(stats_ref)

    w = w_ref[...]
    stats_ref[0:1, :] = stats_ref[0:1, :] + jnp.sum(w, axis=0, keepdims=True)
    stats_ref[1:2, :] = stats_ref[1:2, :] + jnp.sum(w * w, axis=0, keepdims=True)


def _tc4a(w):
    nb = 80
    bs = E // nb
    return pl.pallas_call(
        _tc4a_body,
        grid=(nb,),
        in_specs=[pl.BlockSpec((bs, EW), lambda i: (i, 0))],
        out_specs=pl.BlockSpec((8, EW), lambda i: (0, 0)),
        out_shape=jax.ShapeDtypeStruct((8, EW), f32),
    )(w)


def _tc4b_body(w_ref, ow_ref, stats_ref, g_ref, b_ref, wa3_ref, wa4_ref, out_ref):
    inv_e = 1.0 / E
    mean = stats_ref[0:1, :] * inv_e
    var = stats_ref[1:2, :] * inv_e - mean * mean
    rstd = lax.rsqrt(var + 1e-5)
    wn = (w_ref[...] - mean) * rstd * g_ref[...] + b_ref[...]
    out_ref[...] = (jnp.dot(wn, wa3_ref[...], preferred_element_type=f32)
                    + jnp.dot(ow_ref[...], wa4_ref[...], preferred_element_type=f32))


def _tc4b(w, ow, stats, gamma, beta, Wa3, Wa4):
    nb = 80
    bs = E // nb
    return pl.pallas_call(
        _tc4b_body,
        grid=(nb,),
        in_specs=[
            pl.BlockSpec((bs, EW), lambda i: (i, 0)),
            pl.BlockSpec((bs, EW), lambda i: (i, 0)),
            pl.BlockSpec((8, EW), lambda i: (0, 0)),
            pl.BlockSpec((1, EW), lambda i: (0, 0)),
            pl.BlockSpec((1, EW), lambda i: (0, 0)),
            pl.BlockSpec((EW, EW), lambda i: (0, 0)),
            pl.BlockSpec((EW, EW), lambda i: (0, 0)),
        ],
        out_specs=pl.BlockSpec((bs, EW), lambda i: (i, 0)),
        out_shape=jax.ShapeDtypeStruct((E, EW), f32),
    )(w, ow, stats, gamma, beta, Wa3, Wa4)


def _wid():
    return lax.axis_index("c") * NS + lax.axis_index("s")


def _sc_edge_logits_body(has_eterm, *refs):
    if has_eterm:
        (taba_hbm, tabb_hbm, src_hbm, dst_hbm, et_hbm, eexp_hbm, sparts_hbm,
         taba_v, tabb_v, src_v, dst_v, et_v, ee_v, sloc_v) = refs
    else:
        (taba_hbm, tabb_hbm, src_hbm, dst_hbm, eexp_hbm, sparts_hbm,
         taba_v, tabb_v, src_v, dst_v, ee_v, sloc_v) = refs
        et_v = None
    wid = _wid()
    base = wid * EPW
    pltpu.sync_copy(taba_hbm, taba_v)
    pltpu.sync_copy(tabb_hbm, tabb_v)
    pltpu.sync_copy(src_hbm.at[pl.ds(base, EPW)], src_v)
    pltpu.sync_copy(dst_hbm.at[pl.ds(base, EPW)], dst_v)
    if has_eterm:
        pltpu.sync_copy(et_hbm.at[pl.ds(base, EPW)], et_v)

    zf16 = jnp.zeros((16,), f32)

    def zloop(r, _):
        sloc_v[pl.ds(r * 16, 16)] = zf16
        return 0

    lax.fori_loop(0, RPT, zloop, 0)

    def eloop(j, _):
        sl = pl.ds(j * 16, 16)
        sv = src_v[sl]
        dv = dst_v[sl]
        av = plsc.load_gather(taba_v, [sv])
        bv = plsc.load_gather(tabb_v, [dv])
        x = av + bv
        if has_eterm:
            x = x + et_v[sl]
        ev = jnp.exp(_lrelu(x))
        ee_v[sl] = ev
        plsc.addupdate_scatter(sloc_v, [dv], ev)
        return 0

    lax.fori_loop(0, EPW // 16, eloop, 0)

    pltpu.sync_copy(ee_v, eexp_hbm.at[pl.ds(base, EPW)])
    pltpu.sync_copy(sloc_v, sparts_hbm.at[wid])


def _sc_edge_logits(taba, tabb, src, dst, eterm=None):
    has_eterm = eterm is not None
    scratch = [
        pltpu.VMEM((N,), f32),
        pltpu.VMEM((N,), f32),
        pltpu.VMEM((EPW,), i32),
        pltpu.VMEM((EPW,), i32),
    ]
    if has_eterm:
        scratch.append(pltpu.VMEM((EPW,), f32))
    scratch += [
        pltpu.VMEM((EPW,), f32),
        pltpu.VMEM((N,), f32),
    ]
    fn = pl.kernel(
        functools.partial(_sc_edge_logits_body, has_eterm),
        out_type=[
            jax.ShapeDtypeStruct((E,), f32),
            jax.ShapeDtypeStruct((NWORK, N), f32),
        ],
        mesh=_MESH,
        compiler_params=_SC_PARAMS,
        scratch_types=scratch,
    )
    if has_eterm:
        return fn(taba, tabb, src, dst, eterm)
    return fn(taba, tabb, src, dst)


RT = 640
RCH = 80


def _sc_msgpass_body(src_hbm, dst_hbm, ee_hbm, sparts_hbm, tab_hbm, zh_hbm,
                     partsH_hbm, recip_hbm,
                     srcb_v, dstb2_v, eb_v, rows_v, spb_v, rb_v,
                     sem_i, sem_g, sem_s, accH_s):
    c = lax.axis_index("c")
    sidx = lax.axis_index("s")
    wid = c * NS + sidx
    tbase = sidx * RPT

    pltpu.sync_copy(zh_hbm, accH_s.at[pl.ds(tbase, RPT)])

    @pl.when(c == 0)
    def _():
        rbase = sidx * RT

        def ch(ci, _):
            cb = rbase + ci * RCH

            @pl.when(cb < N)
            def _():
                pltpu.sync_copy(sparts_hbm.at[:, pl.ds(cb, RCH)], spb_v)

                def gl(g, _):
                    sl = pl.ds(g * 16, 16)
                    acc = spb_v[0, sl]
                    for p in range(1, NWORK):
                        acc = acc + spb_v[p, sl]
                    rb_v[sl] = 1.0 / jnp.maximum(acc, 1e-30)
                    return 0

                lax.fori_loop(0, RCH // 16, gl, 0)
                pltpu.sync_copy(rb_v, recip_hbm.at[pl.ds(cb, RCH)])

            return 0

        lax.fori_loop(0, RT // RCH, ch, 0)

    plsc.subcore_barrier()

    def sup(i, _):
        def gather_cp(g, buf):
            return pltpu.async_copy(tab_hbm.at[srcb_v.at[pl.ds(g * K, K)]],
                                    rows_v.at[buf], sem_g)

        def scatter_cp(g, buf):
            return pltpu.async_copy(rows_v.at[buf],
                                    accH_s.at[dstb2_v.at[g]], sem_s, add=True)

        base = wid * EPW + i * SUP
        loads = [
            pltpu.async_copy(src_hbm.at[pl.ds(base, SUP)], srcb_v, sem_i),
            pltpu.async_copy(ee_hbm.at[pl.ds(base, SUP)], eb_v, sem_i),
        ] + [
            pltpu.async_copy(dst_hbm.at[pl.ds(base + g * K, K)],
                             dstb2_v.at[g], sem_i)
            for g in range(G)
        ]
        for cp in loads:
            cp.wait()

        gds = {0: gather_cp(0, 0)}
        sds = {}

        for g in range(G):
            buf = g % 3
            if g >= 2:
                sds[g - 2].wait()
            if g + 1 < G:
                gds[g + 1] = gather_cp(g + 1, (g + 1) % 3)
            gds[g].wait()

            def scale(j, _):
                av = eb_v[pl.ds(g * K + j * 16, 16)]
                for l in range(16):
                    a = av[l]
                    e = j * 16 + l
                    for q in range(D // 16):
                        sl = pl.ds(q * 16, 16)
                        rows_v[buf, e, sl] = rows_v[buf, e, sl] * a
                return 0

            lax.fori_loop(0, K // 16, scale, 0)
            sds[g] = scatter_cp(g, buf)
        sds[G - 2].wait()
        sds[G - 1].wait()
        return 0

    lax.fori_loop(0, NSUP, sup, 0)
    plsc.subcore_barrier()

    pltpu.sync_copy(accH_s.at[pl.ds(tbase, RPT)],
                    partsH_hbm.at[c, pl.ds(tbase, RPT)])


def _sc_msgpass(src, dst, eexp, sparts, tab):
    fn = pl.kernel(
        _sc_msgpass_body,
        out_type=[
            jax.ShapeDtypeStruct((NC, N, D), f32),
            jax.ShapeDtypeStruct((N,), f32),
        ],
        mesh=_MESH,
        compiler_params=_SC_PARAMS,
        scratch_types=[
            pltpu.VMEM((SUP,), i32),
            pltpu.VMEM((G, K), i32),
            pltpu.VMEM((SUP,), f32),
            pltpu.VMEM((3, K, D), f32),
            pltpu.VMEM((NWORK, RCH), f32),
            pltpu.VMEM((RCH,), f32),
            pltpu.SemaphoreType.DMA,
            pltpu.SemaphoreType.DMA,
            pltpu.SemaphoreType.DMA,
            pltpu.VMEM_SHARED((N, D), f32),
        ],
    )
    return fn(src, dst, eexp, sparts, tab, jnp.zeros((RPT, D), f32))


def _sc_wacc_body(dst_hbm, ee_hbm, w_hbm, zw_hbm, partsW_hbm,
                  dstb2_v, eb_v, wrows_v,
                  sem_i, sem_s, accW_s):
    c = lax.axis_index("c")
    sidx = lax.axis_index("s")
    wid = c * NS + sidx
    tbase = sidx * RPT

    pltpu.sync_copy(zw_hbm, accW_s.at[pl.ds(tbase, RPT)])
    plsc.subcore_barrier()

    def sup(i, _):
        base = wid * EPW + i * SUP
        loads = [
            pltpu.async_copy(ee_hbm.at[pl.ds(base, SUP)], eb_v, sem_i),
            pltpu.async_copy(w_hbm.at[pl.ds(base, SUP)], wrows_v, sem_i),
        ] + [
            pltpu.async_copy(dst_hbm.at[pl.ds(base + g * K, K)],
                             dstb2_v.at[g], sem_i)
            for g in range(G)
        ]
        for cp in loads:
            cp.wait()

        def wscale(j, _):
            av = eb_v[pl.ds(j * 16, 16)]
            for l in range(16):
                e = j * 16 + l
                sl = pl.ds(0, 16)
                wrows_v[e, sl] = wrows_v[e, sl] * av[l]
            return 0

        lax.fori_loop(0, SUP // 16, wscale, 0)

        scatters = [
            pltpu.async_copy(wrows_v.at[pl.ds(g * K, K)],
                             accW_s.at[dstb2_v.at[g]], sem_s, add=True)
            for g in range(G)
        ]
        for cp in scatters:
            cp.wait()
        return 0

    lax.fori_loop(0, NSUP, sup, 0)
    plsc.subcore_barrier()

    pltpu.sync_copy(accW_s.at[pl.ds(tbase, RPT)],
                    partsW_hbm.at[c, pl.ds(tbase, RPT)])


def _sc_wacc(dst, eexp, w):
    fn = pl.kernel(
        _sc_wacc_body,
        out_type=jax.ShapeDtypeStruct((NC, N, EW), f32),
        mesh=_MESH,
        compiler_params=_SC_PARAMS,
        scratch_types=[
            pltpu.VMEM((G, K), i32),
            pltpu.VMEM((SUP,), f32),
            pltpu.VMEM((SUP, EW), f32),
            pltpu.SemaphoreType.DMA,
            pltpu.SemaphoreType.DMA,
            pltpu.VMEM_SHARED((N, EW), f32),
        ],
    )
    return fn(dst, eexp, w, jnp.zeros((RPT, EW), f32))


def _sc_wout_body(src_hbm, dst_hbm, p1_hbm, p2_hbm, wc_hbm, out_hbm,
                  srcb_v, dstb_v, g1_v, g2_v, wcb_v, sem_i, sem_g):
    wid = _wid()

    def sup(i, _):
        base = wid * EPW + i * SUP
        loads = [
            pltpu.async_copy(src_hbm.at[pl.ds(base, SUP)], srcb_v, sem_i),
            pltpu.async_copy(dst_hbm.at[pl.ds(base, SUP)], dstb_v, sem_i),
            pltpu.async_copy(wc_hbm.at[pl.ds(base, SUP)], wcb_v, sem_i),
        ]
        for cp in loads:
            cp.wait()
        gathers = [
            pltpu.async_copy(p1_hbm.at[srcb_v.at[pl.ds(g * K, K)]],
                             g1_v.at[pl.ds(g * K, K)], sem_g)
            for g in range(G)
        ] + [
            pltpu.async_copy(p2_hbm.at[dstb_v.at[pl.ds(g * K, K)]],
                             g2_v.at[pl.ds(g * K, K)], sem_g)
            for g in range(G)
        ]
        for cp in gathers:
            cp.wait()

        def add(e, _):
            sl = pl.ds(0, 16)
            wcb_v[e, sl] = wcb_v[e, sl] + g1_v[e, sl] + g2_v[e, sl]
            return 0

        lax.fori_loop(0, SUP, add, 0)
        pltpu.sync_copy(wcb_v, out_hbm.at[pl.ds(base, SUP)])
        return 0

    lax.fori_loop(0, NSUP, sup, 0)


def _sc_wout(src, dst, p1, p2, wc):
    fn = pl.kernel(
        _sc_wout_body,
        out_type=jax.ShapeDtypeStruct((E, EW), f32),
        mesh=_MESH,
        compiler_params=_SC_PARAMS,
        scratch_types=[
            pltpu.VMEM((SUP,), i32),
            pltpu.VMEM((SUP,), i32),
            pltpu.VMEM((SUP, EW), f32),
            pltpu.VMEM((SUP, EW), f32),
            pltpu.VMEM((SUP, EW), f32),
            pltpu.SemaphoreType.DMA,
            pltpu.SemaphoreType.DMA,
        ],
    )
    return fn(src, dst, p1, p2, wc)


def kernel(h, edge_index, edge_w, W_embed_node, W_nsa, b_nsa, W_embed_edge,
           W_esa, W_attn, W_conc, b_conc, W_aggre, bn_gamma, bn_beta):
    src = edge_index[0]
    dst = edge_index[1]

    Wnsa2 = jnp.concatenate([W_nsa[:D], W_nsa[D:]], axis=1)
    bias2 = jnp.concatenate([jnp.zeros((1,), f32), b_nsa]).reshape(1, 2)
    hn, asad = _tc1a(h, W_embed_node, Wnsa2, bias2)
    w, wa3 = _tc1b(edge_w, W_embed_edge, W_attn[2 * D:2 * D + EW])

    eexp1, sparts1 = _sc_edge_logits(asad[:, 0], asad[:, 1], src, dst)
    partsH1, recip1 = _sc_msgpass(src, dst, eexp1, sparts1, hn)

    Wat12 = jnp.concatenate([W_attn[:D], W_attn[D:2 * D]], axis=1)
    h1, b12 = _tc2(partsH1, recip1.reshape(N, 1), Wat12)

    eexp2, sparts2 = _sc_edge_logits(b12[:, 0], b12[:, 1], src, dst,
                                     wa3.reshape(E))
    pH, recip2 = _sc_msgpass(src, dst, eexp2, sparts2, h1)
    pW = _sc_wacc(dst, eexp2, w)

    h2, p1, p2 = _tc3(
        pH, pW, recip2.reshape(N, 1), hn,
        W_conc[:D], W_conc[D:D + EW], W_conc[D + EW:],
        b_conc.reshape(1, D),
        W_aggre[:D], W_aggre[D:2 * D])

    stats = _tc4a(w)
    wc = _tc4b(w, edge_w, stats, bn_gamma.reshape(1, EW), bn_beta.reshape(1, EW),
               W_aggre[2 * D:2 * D + EW], W_aggre[2 * D + EW:])

    w_out = _sc_wout(src, dst, p1, p2, wc)
    return h2, w_out


# speedup vs baseline: 1.0071x; 1.0071x over previous
import functools
import jax
import jax.numpy as jnp
from jax import lax
from jax.experimental import pallas as pl
from jax.experimental.pallas import tpu as pltpu
from jax.experimental.pallas import tpu_sc as plsc

N = 10000
E = 320000
D = 128
EW = 16

NC = 2
NS = 16
NWORK = NC * NS
EPW = E // NWORK
K = 80
G = 5
SUP = G * K
NSUP = EPW // SUP
RPT = N // NS

f32 = jnp.float32
i32 = jnp.int32

_MESH = plsc.VectorSubcoreMesh(core_axis_name="c", subcore_axis_name="s")
_SC_PARAMS = pltpu.CompilerParams(needs_layout_passes=False, use_tc_tiling_on_sc=False)


def _lrelu(x):
    return jnp.maximum(x, 0.1 * x)


def _tc1a_body(h_ref, wen_ref, wnsa_ref, bias_ref, hn_ref, asad_ref):
    hn = jnp.dot(h_ref[...], wen_ref[...], preferred_element_type=f32)
    hn_ref[...] = hn
    asad_ref[...] = jnp.dot(hn, wnsa_ref[...], preferred_element_type=f32) + bias_ref[...]


def _tc1a(h, Wen, Wnsa2, bias2):
    nb = 10
    bs = N // nb
    return pl.pallas_call(
        _tc1a_body,
        grid=(nb,),
        in_specs=[
            pl.BlockSpec((bs, D), lambda i: (i, 0)),
            pl.BlockSpec((D, D), lambda i: (0, 0)),
            pl.BlockSpec((D, 2), lambda i: (0, 0)),
            pl.BlockSpec((1, 2), lambda i: (0, 0)),
        ],
        out_specs=[
            pl.BlockSpec((bs, D), lambda i: (i, 0)),
            pl.BlockSpec((bs, 2), lambda i: (i, 0)),
        ],
        out_shape=[
            jax.ShapeDtypeStruct((N, D), f32),
            jax.ShapeDtypeStruct((N, 2), f32),
        ],
    )(h, Wen, Wnsa2, bias2)


def _tc1b_body(ew_ref, wee_ref, wa3_ref, w_ref, wa3o_ref):
    w = jnp.dot(ew_ref[...], wee_ref[...], preferred_element_type=f32)
    w_ref[...] = w
    wa3o_ref[...] = jnp.dot(w, wa3_ref[...], preferred_element_type=f32)


def _tc1b(edge_w, Wee, Wa3):
    nb = 80
    bs = E // nb
    return pl.pallas_call(
        _tc1b_body,
        grid=(nb,),
        in_specs=[
            pl.BlockSpec((bs, EW), lambda i: (i, 0)),
            pl.BlockSpec((EW, EW), lambda i: (0, 0)),
            pl.BlockSpec((EW, 1), lambda i: (0, 0)),
        ],
        out_specs=[
            pl.BlockSpec((bs, EW), lambda i: (i, 0)),
            pl.BlockSpec((bs, 1), lambda i: (i, 0)),
        ],
        out_shape=[
            jax.ShapeDtypeStruct((E, EW), f32),
            jax.ShapeDtypeStruct((E, 1), f32),
        ],
    )(edge_w, Wee, Wa3)


def _tc2_body(parts_ref, recip_ref, wat_ref, h1_ref, b12_ref):
    h1 = (parts_ref[0] + parts_ref[1]) * recip_ref[...]
    h1_ref[...] = h1
    b12_ref[...] = jnp.dot(h1, wat_ref[...], preferred_element_type=f32)


def _tc2(parts, recip, Wat12):
    nb = 10
    bs = N // nb
    return pl.pallas_call(
        _tc2_body,
        grid=(nb,),
        in_specs=[
            pl.BlockSpec((NC, bs, D), lambda i: (0, i, 0)),
            pl.BlockSpec((bs, 1), lambda i: (i, 0)),
            pl.BlockSpec((D, 2), lambda i: (0, 0)),
        ],
        out_specs=[
            pl.BlockSpec((bs, D), lambda i: (i, 0)),
            pl.BlockSpec((bs, 2), lambda i: (i, 0)),
        ],
        out_shape=[
            jax.ShapeDtypeStruct((N, D), f32),
            jax.ShapeDtypeStruct((N, 2), f32),
        ],
    )(parts, recip, Wat12)


def _tc3_body(ph_ref, pw_ref, recip_ref, hn_ref, wc1_ref, wc2_ref,
              wc3_ref, bc_ref, wa1_ref, wa2_ref, h2_ref, p1_ref, p2_ref):
    r = recip_ref[...]
    haggH = (ph_ref[0] + ph_ref[1]) * r
    haggW = (pw_ref[0] + pw_ref[1]) * r
    h2 = (jnp.dot(haggH, wc1_ref[...], preferred_element_type=f32)
          + jnp.dot(haggW, wc2_ref[...], preferred_element_type=f32)
          + jnp.dot(hn_ref[...], wc3_ref[...], preferred_element_type=f32)
          + bc_ref[...])
    h2_ref[...] = h2
    p1_ref[...] = jnp.dot(h2, wa1_ref[...], preferred_element_type=f32)
    p2_ref[...] = jnp.dot(h2, wa2_ref[...], preferred_element_type=f32)


def _tc3(ph, pw, recip, hn, Wc1, Wc2, Wc3, bc, Wa1, Wa2):
    nb = 10
    bs = N // nb
    return pl.pallas_call(
        _tc3_body,
        grid=(nb,),
        in_specs=[
            pl.BlockSpec((NC, bs, D), lambda i: (0, i, 0)),
            pl.BlockSpec((NC, bs, EW), lambda i: (0, i, 0)),
            pl.BlockSpec((bs, 1), lambda i: (i, 0)),
            pl.BlockSpec((bs, D), lambda i: (i, 0)),
            pl.BlockSpec((D, D), lambda i: (0, 0)),
            pl.BlockSpec((EW, D), lambda i: (0, 0)),
            pl.BlockSpec((D, D), lambda i: (0, 0)),
            pl.BlockSpec((1, D), lambda i: (0, 0)),
            pl.BlockSpec((D, EW), lambda i: (0, 0)),
            pl.BlockSpec((D, EW), lambda i: (0, 0)),
        ],
        out_specs=[
            pl.BlockSpec((bs, D), lambda i: (i, 0)),
            pl.BlockSpec((bs, EW), lambda i: (i, 0)),
            pl.BlockSpec((bs, EW), lambda i: (i, 0)),
        ],
        out_shape=[
            jax.ShapeDtypeStruct((N, D), f32),
            jax.ShapeDtypeStruct((N, EW), f32),
            jax.ShapeDtypeStruct((N, EW), f32),
        ],
    )(ph, pw, recip, hn, Wc1, Wc2, Wc3, bc, Wa1, Wa2)


def _tc4a_body(w_ref, stats_ref):
    i = pl.program_id(0)

    @pl.when(i == 0)
    def _():
        stats_ref[...] = jnp.zeros_like(stats_ref)

    w = w_ref[...]
    stats_ref[0:1, :] = stats_ref[0:1, :] + jnp.sum(w, axis=0, keepdims=True)
    stats_ref[1:2, :] = stats_ref[1:2, :] + jnp.sum(w * w, axis=0, keepdims=True)


def _tc4a(w):
    nb = 80
    bs = E // nb
    return pl.pallas_call(
        _tc4a_body,
        grid=(nb,),
        in_specs=[pl.BlockSpec((bs, EW), lambda i: (i, 0))],
        out_specs=pl.BlockSpec((8, EW), lambda i: (0, 0)),
        out_shape=jax.ShapeDtypeStruct((8, EW), f32),
    )(w)


def _tc4b_body(w_ref, ow_ref, stats_ref, g_ref, b_ref, wa3_ref, wa4_ref, out_ref):
    inv_e = 1.0 / E
    mean = stats_ref[0:1, :] * inv_e
    var = stats_ref[1:2, :] * inv_e - mean * mean
    rstd = lax.rsqrt(var + 1e-5)
    wn = (w_ref[...] - mean) * rstd * g_ref[...] + b_ref[...]
    out_ref[...] = (jnp.dot(wn, wa3_ref[...], preferred_element_type=f32)
                    + jnp.dot(ow_ref[...], wa4_ref[...], preferred_element_type=f32))


def _tc4b(w, ow, stats, gamma, beta, Wa3, Wa4):
    nb = 80
    bs = E // nb
    return pl.pallas_call(
        _tc4b_body,
        grid=(nb,),
        in_specs=[
            pl.BlockSpec((bs, EW), lambda i: (i, 0)),
            pl.BlockSpec((bs, EW), lambda i: (i, 0)),
            pl.BlockSpec((8, EW), lambda i: (0, 0)),
            pl.BlockSpec((1, EW), lambda i: (0, 0)),
            pl.BlockSpec((1, EW), lambda i: (0, 0)),
            pl.BlockSpec((EW, EW), lambda i: (0, 0)),
            pl.BlockSpec((EW, EW), lambda i: (0, 0)),
        ],
        out_specs=pl.BlockSpec((bs, EW), lambda i: (i, 0)),
        out_shape=jax.ShapeDtypeStruct((E, EW), f32),
    )(w, ow, stats, gamma, beta, Wa3, Wa4)


def _wid():
    return lax.axis_index("c") * NS + lax.axis_index("s")


def _sc_edge_logits_body(has_eterm, *refs):
    if has_eterm:
        (taba_hbm, tabb_hbm, eidx_hbm, et_hbm, eexp_hbm, sparts_hbm,
         taba_v, tabb_v, src_v, dst_v, et_v, ee_v, sloc_v) = refs
    else:
        (taba_hbm, tabb_hbm, eidx_hbm, eexp_hbm, sparts_hbm,
         taba_v, tabb_v, src_v, dst_v, ee_v, sloc_v) = refs
        et_v = None
    wid = _wid()
    base = wid * EPW
    pltpu.sync_copy(taba_hbm, taba_v)
    pltpu.sync_copy(tabb_hbm, tabb_v)
    pltpu.sync_copy(eidx_hbm.at[0, pl.ds(base, EPW)], src_v)
    pltpu.sync_copy(eidx_hbm.at[1, pl.ds(base, EPW)], dst_v)
    if has_eterm:
        pltpu.sync_copy(et_hbm.at[pl.ds(base, EPW)], et_v)

    zf16 = jnp.zeros((16,), f32)

    def zloop(r, _):
        sloc_v[pl.ds(r * 16, 16)] = zf16
        return 0

    lax.fori_loop(0, RPT, zloop, 0)

    def eloop(j, _):
        sl = pl.ds(j * 16, 16)
        sv = src_v[sl]
        dv = dst_v[sl]
        av = plsc.load_gather(taba_v, [sv])
        bv = plsc.load_gather(tabb_v, [dv])
        x = av + bv
        if has_eterm:
            x = x + et_v[sl]
        ev = jnp.exp(_lrelu(x))
        ee_v[sl] = ev
        plsc.addupdate_scatter(sloc_v, [dv], ev)
        return 0

    lax.fori_loop(0, EPW // 16, eloop, 0)

    pltpu.sync_copy(ee_v, eexp_hbm.at[pl.ds(base, EPW)])
    pltpu.sync_copy(sloc_v, sparts_hbm.at[wid])


def _sc_edge_logits(taba, tabb, eidx, eterm=None):
    has_eterm = eterm is not None
    scratch = [
        pltpu.VMEM((N,), f32),
        pltpu.VMEM((N,), f32),
        pltpu.VMEM((EPW,), i32),
        pltpu.VMEM((EPW,), i32),
    ]
    if has_eterm:
        scratch.append(pltpu.VMEM((EPW,), f32))
    scratch += [
        pltpu.VMEM((EPW,), f32),
        pltpu.VMEM((N,), f32),
    ]
    fn = pl.kernel(
        functools.partial(_sc_edge_logits_body, has_eterm),
        out_type=[
            jax.ShapeDtypeStruct((E,), f32),
            jax.ShapeDtypeStruct((NWORK, N), f32),
        ],
        mesh=_MESH,
        compiler_params=_SC_PARAMS,
        scratch_types=scratch,
    )
    if has_eterm:
        return fn(taba, tabb, eidx, eterm)
    return fn(taba, tabb, eidx)


RT = 640
RCH = 80


def _sc_msgpass_body(eidx_hbm, ee_hbm, sparts_hbm, tab_hbm, zh_hbm,
                     partsH_hbm, recip_hbm,
                     srcb_v, dstb2_v, eb_v, rows_v, spb_v, rb_v,
                     sem_i, sem_g, sem_s, accH_s):
    c = lax.axis_index("c")
    sidx = lax.axis_index("s")
    wid = c * NS + sidx
    tbase = sidx * RPT

    pltpu.sync_copy(zh_hbm, accH_s.at[pl.ds(tbase, RPT)])

    @pl.when(c == 0)
    def _():
        rbase = sidx * RT

        def ch(ci, _):
            cb = rbase + ci * RCH

            @pl.when(cb < N)
            def _():
                pltpu.sync_copy(sparts_hbm.at[:, pl.ds(cb, RCH)], spb_v)

                def gl(g, _):
                    sl = pl.ds(g * 16, 16)
                    acc = spb_v[0, sl]
                    for p in range(1, NWORK):
                        acc = acc + spb_v[p, sl]
                    rb_v[sl] = 1.0 / jnp.maximum(acc, 1e-30)
                    return 0

                lax.fori_loop(0, RCH // 16, gl, 0)
                pltpu.sync_copy(rb_v, recip_hbm.at[pl.ds(cb, RCH)])

            return 0

        lax.fori_loop(0, RT // RCH, ch, 0)

    plsc.subcore_barrier()

    def sup(i, _):
        def gather_cp(g, buf):
            return pltpu.async_copy(tab_hbm.at[srcb_v.at[pl.ds(g * K, K)]],
                                    rows_v.at[buf], sem_g)

        def scatter_cp(g, buf):
            return pltpu.async_copy(rows_v.at[buf],
                                    accH_s.at[dstb2_v.at[g]], sem_s, add=True)

        base = wid * EPW + i * SUP
        loads = [
            pltpu.async_copy(eidx_hbm.at[0, pl.ds(base, SUP)], srcb_v, sem_i),
            pltpu.async_copy(ee_hbm.at[pl.ds(base, SUP)], eb_v, sem_i),
        ] + [
            pltpu.async_copy(eidx_hbm.at[1, pl.ds(base + g * K, K)],
                             dstb2_v.at[g], sem_i)
            for g in range(G)
        ]
        for cp in loads:
            cp.wait()

        gds = {0: gather_cp(0, 0)}
        sds = {}

        for g in range(G):
            buf = g % 3
            if g >= 2:
                sds[g - 2].wait()
            if g + 1 < G:
                gds[g + 1] = gather_cp(g + 1, (g + 1) % 3)
            gds[g].wait()

            def scale(j, _):
                av = eb_v[pl.ds(g * K + j * 16, 16)]
                for l in range(16):
                    a = av[l]
                    e = j * 16 + l
                    for q in range(D // 16):
                        sl = pl.ds(q * 16, 16)
                        rows_v[buf, e, sl] = rows_v[buf, e, sl] * a
                return 0

            lax.fori_loop(0, K // 16, scale, 0)
            sds[g] = scatter_cp(g, buf)
        sds[G - 2].wait()
        sds[G - 1].wait()
        return 0

    lax.fori_loop(0, NSUP, sup, 0)
    plsc.subcore_barrier()

    pltpu.sync_copy(accH_s.at[pl.ds(tbase, RPT)],
                    partsH_hbm.at[c, pl.ds(tbase, RPT)])


def _sc_msgpass(eidx, eexp, sparts, tab):
    fn = pl.kernel(
        _sc_msgpass_body,
        out_type=[
            jax.ShapeDtypeStruct((NC, N, D), f32),
            jax.ShapeDtypeStruct((N,), f32),
        ],
        mesh=_MESH,
        compiler_params=_SC_PARAMS,
        scratch_types=[
            pltpu.VMEM((SUP,), i32),
            pltpu.VMEM((G, K), i32),
            pltpu.VMEM((SUP,), f32),
            pltpu.VMEM((3, K, D), f32),
            pltpu.VMEM((NWORK, RCH), f32),
            pltpu.VMEM((RCH,), f32),
            pltpu.SemaphoreType.DMA,
            pltpu.SemaphoreType.DMA,
            pltpu.SemaphoreType.DMA,
            pltpu.VMEM_SHARED((N, D), f32),
        ],
    )
    return fn(eidx, eexp, sparts, tab, jnp.zeros((RPT, D), f32))


def _sc_wacc_body(eidx_hbm, ee_hbm, w_hbm, zw_hbm, partsW_hbm,
                  dstb2_v, eb_v, wrows_v,
                  sem_i, sem_s, accW_s):
    c = lax.axis_index("c")
    sidx = lax.axis_index("s")
    wid = c * NS + sidx
    tbase = sidx * RPT

    pltpu.sync_copy(zw_hbm, accW_s.at[pl.ds(tbase, RPT)])
    plsc.subcore_barrier()

    def sup(i, _):
        base = wid * EPW + i * SUP
        loads = [
            pltpu.async_copy(ee_hbm.at[pl.ds(base, SUP)], eb_v, sem_i),
            pltpu.async_copy(w_hbm.at[pl.ds(base, SUP)], wrows_v, sem_i),
        ] + [
            pltpu.async_copy(eidx_hbm.at[1, pl.ds(base + g * K, K)],
                             dstb2_v.at[g], sem_i)
            for g in range(G)
        ]
        for cp in loads:
            cp.wait()

        def wscale(j, _):
            av = eb_v[pl.ds(j * 16, 16)]
            for l in range(16):
                e = j * 16 + l
                sl = pl.ds(0, 16)
                wrows_v[e, sl] = wrows_v[e, sl] * av[l]
            return 0

        lax.fori_loop(0, SUP // 16, wscale, 0)

        scatters = [
            pltpu.async_copy(wrows_v.at[pl.ds(g * K, K)],
                             accW_s.at[dstb2_v.at[g]], sem_s, add=True)
            for g in range(G)
        ]
        for cp in scatters:
            cp.wait()
        return 0

    lax.fori_loop(0, NSUP, sup, 0)
    plsc.subcore_barrier()

    pltpu.sync_copy(accW_s.at[pl.ds(tbase, RPT)],
                    partsW_hbm.at[c, pl.ds(tbase, RPT)])


def _sc_wacc(eidx, eexp, w):
    fn = pl.kernel(
        _sc_wacc_body,
        out_type=jax.ShapeDtypeStruct((NC, N, EW), f32),
        mesh=_MESH,
        compiler_params=_SC_PARAMS,
        scratch_types=[
            pltpu.VMEM((G, K), i32),
            pltpu.VMEM((SUP,), f32),
            pltpu.VMEM((SUP, EW), f32),
            pltpu.SemaphoreType.DMA,
            pltpu.SemaphoreType.DMA,
            pltpu.VMEM_SHARED((N, EW), f32),
        ],
    )
    return fn(eidx, eexp, w, jnp.zeros((RPT, EW), f32))


def _sc_wout_body(eidx_hbm, p1_hbm, p2_hbm, wc_hbm, out_hbm,
                  srcb_v, dstb_v, g1_v, g2_v, wcb_v, sem_i, sem_g):
    wid = _wid()

    def sup(i, _):
        base = wid * EPW + i * SUP
        loads = [
            pltpu.async_copy(eidx_hbm.at[0, pl.ds(base, SUP)], srcb_v, sem_i),
            pltpu.async_copy(eidx_hbm.at[1, pl.ds(base, SUP)], dstb_v, sem_i),
            pltpu.async_copy(wc_hbm.at[pl.ds(base, SUP)], wcb_v, sem_i),
        ]
        for cp in loads:
            cp.wait()
        gathers = [
            pltpu.async_copy(p1_hbm.at[srcb_v.at[pl.ds(g * K, K)]],
                             g1_v.at[pl.ds(g * K, K)], sem_g)
            for g in range(G)
        ] + [
            pltpu.async_copy(p2_hbm.at[dstb_v.at[pl.ds(g * K, K)]],
                             g2_v.at[pl.ds(g * K, K)], sem_g)
            for g in range(G)
        ]
        for cp in gathers:
            cp.wait()

        def add(e, _):
            sl = pl.ds(0, 16)
            wcb_v[e, sl] = wcb_v[e, sl] + g1_v[e, sl] + g2_v[e, sl]
            return 0

        lax.fori_loop(0, SUP, add, 0)
        pltpu.sync_copy(wcb_v, out_hbm.at[pl.ds(base, SUP)])
        return 0

    lax.fori_loop(0, NSUP, sup, 0)


def _sc_wout(eidx, p1, p2, wc):
    fn = pl.kernel(
        _sc_wout_body,
        out_type=jax.ShapeDtypeStruct((E, EW), f32),
        mesh=_MESH,
        compiler_params=_SC_PARAMS,
        scratch_types=[
            pltpu.VMEM((SUP,), i32),
            pltpu.VMEM((SUP,), i32),
            pltpu.VMEM((SUP, EW), f32),
            pltpu.VMEM((SUP, EW), f32),
            pltpu.VMEM((SUP, EW), f32),
            pltpu.SemaphoreType.DMA,
            pltpu.SemaphoreType.DMA,
        ],
    )
    return fn(eidx, p1, p2, wc)


def kernel(h, edge_index, edge_w, W_embed_node, W_nsa, b_nsa, W_embed_edge,
           W_esa, W_attn, W_conc, b_conc, W_aggre, bn_gamma, bn_beta):
    Wnsa2 = jnp.concatenate([W_nsa[:D], W_nsa[D:]], axis=1)
    bias2 = jnp.concatenate([jnp.zeros((1,), f32), b_nsa]).reshape(1, 2)
    hn, asad = _tc1a(h, W_embed_node, Wnsa2, bias2)
    w, wa3 = _tc1b(edge_w, W_embed_edge, W_attn[2 * D:2 * D + EW])

    eexp1, sparts1 = _sc_edge_logits(asad[:, 0], asad[:, 1], edge_index)
    partsH1, recip1 = _sc_msgpass(edge_index, eexp1, sparts1, hn)

    Wat12 = jnp.concatenate([W_attn[:D], W_attn[D:2 * D]], axis=1)
    h1, b12 = _tc2(partsH1, recip1.reshape(N, 1), Wat12)

    eexp2, sparts2 = _sc_edge_logits(b12[:, 0], b12[:, 1], edge_index,
                                     wa3.reshape(E))
    pH, recip2 = _sc_msgpass(edge_index, eexp2, sparts2, h1)
    pW = _sc_wacc(edge_index, eexp2, w)

    h2, p1, p2 = _tc3(
        pH, pW, recip2.reshape(N, 1), hn,
        W_conc[:D], W_conc[D:D + EW], W_conc[D + EW:],
        b_conc.reshape(1, D),
        W_aggre[:D], W_aggre[D:2 * D])

    stats = _tc4a(w)
    wc = _tc4b(w, edge_w, stats, bn_gamma.reshape(1, EW), bn_beta.reshape(1, EW),
               W_aggre[2 * D:2 * D + EW], W_aggre[2 * D + EW:])

    w_out = _sc_wout(edge_index, p1, p2, wc)
    return h2, w_out
